# baseline (speedup 1.0000x reference)
import functools

import jax
import jax.numpy as jnp
from jax import lax
from jax.experimental import pallas as pl
from jax.experimental.pallas import tpu as pltpu
from jax.experimental.pallas import tpu_sc as plsc

N_NODES = 50000
EMB_DIM = 64
N_EDGES = 800000
N_LAYERS = 3

NC = 2
NS = 16
LANES = 16

HALF = N_NODES // NC
ACC_ROWS = 25088
ROWS_PER_TILE = ACC_ROWS // NS
DUMMY = HALF

SCAN_B = 768
FB = 128
PEND = 160
EDGES_PER_TILE = 50688
SCAN_BLOCKS = EDGES_PER_TILE // SCAN_B
E_PAD = NS * EDGES_PER_TILE

DB = 128
DCHUNKS = E_PAD // (NS * DB)
GI_D = 4
DBLOCKS = DCHUNKS // GI_D

CH = 32
Z_CH = 32
SEG = ROWS_PER_TILE

_MESH = plsc.VectorSubcoreMesh(core_axis_name="c", subcore_axis_name="s")
_PARAMS = pltpu.CompilerParams(
    use_tc_tiling_on_sc=False, needs_layout_passes=False)


def _lane_bcast(vec, lane):
    idx = jnp.full((LANES,), lane, jnp.int32)
    return jnp.take_along_axis(vec, idx, axis=0)


def _rsqrt16(d):
    i = lax.bitcast_convert_type(d, jnp.int32)
    i = 0x5F3759DF - jnp.right_shift(i, 1)
    y = lax.bitcast_convert_type(i, jnp.float32)
    for _ in range(3):
        y = y * (1.5 - 0.5 * d * y * y)
    return jnp.where(d >= 0.5, y, 0.0)


def _remap16(t, base):
    local = t - base
    ok = jnp.logical_and(local >= 0, local < HALF)
    return jnp.where(ok, local, DUMMY)


def _deg_kernel_body(to_h, table_h, dinv_h, y0_h,
                     tob_big, hist_v, tmp_seg, seg_deg, dinvbuf, tbuf, ybuf,
                     stage_s):
    c = lax.axis_index("c")
    s = lax.axis_index("s")
    base = c * HALF
    zeros16 = jnp.zeros((LANES,), jnp.float32)
    ones16 = jnp.ones((LANES,), jnp.float32)

    def zfill(i, _):
        hist_v[pl.ds(i * LANES, LANES)] = zeros16
        return 0

    lax.fori_loop(0, ACC_ROWS // LANES, zfill, 0)

    def block_step(nb, _):
        ebase = (s * DCHUNKS + nb * GI_D) * DB
        pltpu.sync_copy(to_h.at[pl.ds(ebase, GI_D * DB)], tob_big)
        for q in range(GI_D * DB // LANES):
            t = tob_big[pl.ds(q * LANES, LANES)]
            plsc.addupdate_scatter(hist_v, [_remap16(t, base)], ones16)
        return 0

    lax.fori_loop(0, DBLOCKS, block_step, 0)

    pltpu.sync_copy(hist_v, stage_s.at[s])
    plsc.subcore_barrier()

    seg0 = s * SEG
    pltpu.sync_copy(stage_s.at[0, pl.ds(seg0, SEG)], seg_deg)
    for t in range(1, NS):
        pltpu.sync_copy(stage_s.at[t, pl.ds(seg0, SEG)], tmp_seg)

        def acc_step(i, _, t=t):
            sl = pl.ds(i * LANES, LANES)
            seg_deg[sl] = seg_deg[sl] + tmp_seg[sl]
            return 0

        lax.fori_loop(0, SEG // LANES, acc_step, 0)

    def scale_step(k, _):
        start_l = lax.min(seg0 + k * CH, HALF - CH)
        off = start_l - seg0
        gstart = base + start_l
        for g in range(CH // LANES):
            d = seg_deg[pl.ds(off + g * LANES, LANES)]
            dinvbuf[pl.ds(g * LANES, LANES)] = _rsqrt16(d)
        pltpu.sync_copy(dinvbuf, dinv_h.at[pl.ds(gstart, CH)])
        pltpu.sync_copy(table_h.at[pl.ds(gstart, CH)], tbuf)

        def group(g, _):
            dvec = dinvbuf[pl.ds(g * LANES, LANES)]
            for r16 in range(LANES):
                d = _lane_bcast(dvec, r16)
                r = g * LANES + r16
                for k2 in range(EMB_DIM // LANES):
                    sl = pl.ds(k2 * LANES, LANES)
                    ybuf[r, sl] = tbuf[r, sl] * d
            return 0

        lax.fori_loop(0, CH // LANES, group, 0)
        pltpu.sync_copy(ybuf, y0_h.at[pl.ds(gstart, CH)])
        return 0

    lax.fori_loop(0, SEG // CH, scale_step, 0)


def _layer_kernel_body(last, from_h, to_h, y_h, s_h, dinv_h, *refs):
    if last:
        (s_out_h, from_blk, to_blk, pf, pidx, gfid0, gfid1, sidx0, sidx1,
         rows0, rows1, zrow, dinvbuf, acc_s, gsem0, gsem1) = refs
        y_out_h = None
    else:
        (s_out_h, y_out_h, from_blk, to_blk, pf, pidx, gfid0, gfid1,
         sidx0, sidx1, rows0, rows1, zrow, dinvbuf, acc_s,
         gsem0, gsem1) = refs
    gfid = (gfid0, gfid1)
    sidx = (sidx0, sidx1)
    rows = (rows0, rows1)
    gsem = (gsem0, gsem1)

    c = lax.axis_index("c")
    s = lax.axis_index("s")
    base = c * HALF
    zeros16 = jnp.zeros((LANES,), jnp.float32)

    def zfill(r, _):
        for k in range(EMB_DIM // LANES):
            zrow[r, pl.ds(k * LANES, LANES)] = zeros16
        return 0

    lax.fori_loop(0, Z_CH, zfill, 0)

    def zcopy(j, _):
        pltpu.sync_copy(zrow,
                        acc_s.at[pl.ds(s * ROWS_PER_TILE + j * Z_CH, Z_CH)])
        return 0

    lax.fori_loop(0, ROWS_PER_TILE // Z_CH, zcopy, 0)
    plsc.subcore_barrier()

    def retire(p):
        pltpu.make_async_copy(y_h.at[gfid[p]], rows[p], gsem[p]).wait()
        pltpu.sync_copy(rows[p], acc_s.at[sidx[p]], add=True)

    def fire_pair(p, fk):
        @pl.when(fk > 0)
        def _():
            retire(1 - p)
        for k in range(FB // LANES):
            sl = pl.ds(k * LANES, LANES)
            gfid[p][sl] = pf[sl]
            sidx[p][sl] = pidx[sl]
        pltpu.async_copy(y_h.at[gfid[p]], rows[p], gsem[p])

    def block_fn(nb, carry):
        cnt, fk, par = carry
        ebase = s * EDGES_PER_TILE + nb * SCAN_B
        pltpu.sync_copy(from_h.at[pl.ds(ebase, SCAN_B)], from_blk)
        pltpu.sync_copy(to_h.at[pl.ds(ebase, SCAN_B)], to_blk)
        for g in range(SCAN_B // LANES):
            sl = pl.ds(g * LANES, LANES)
            f16 = from_blk[sl]
            local = to_blk[sl] - base
            ok = jnp.logical_and(local >= 0, local < HALF)
            plsc.store_compressed(pf.at[pl.ds(cnt, LANES)], f16, mask=ok)
            plsc.store_compressed(pidx.at[pl.ds(cnt, LANES)], local, mask=ok)
            cnt = cnt + plsc.all_reduce_population_count(ok)[0]
            fired = cnt >= FB

            @pl.when(jnp.logical_and(fired, par == 0))
            def _():
                fire_pair(0, fk)

            @pl.when(jnp.logical_and(fired, par == 1))
            def _():
                fire_pair(1, fk)

            @pl.when(fired)
            def _():
                pf[pl.ds(0, LANES)] = pf[pl.ds(FB, LANES)]
                pidx[pl.ds(0, LANES)] = pidx[pl.ds(FB, LANES)]

            cnt = jnp.where(fired, cnt - FB, cnt)
            fk = jnp.where(fired, fk + 1, fk)
            par = jnp.where(fired, 1 - par, par)
        return cnt, fk, par

    cnt, fk, par = lax.fori_loop(
        0, SCAN_BLOCKS, block_fn,
        (jnp.int32(0), jnp.int32(0), jnp.int32(0)))

    iota16 = lax.iota(jnp.int32, LANES)
    for k in range(FB // LANES):
        sl = pl.ds(k * LANES, LANES)
        m = (iota16 + k * LANES) < cnt
        pf[sl] = jnp.where(m, pf[sl], 0)
        pidx[sl] = jnp.where(m, pidx[sl], DUMMY)

    fired = cnt > 0

    @pl.when(jnp.logical_and(fired, par == 0))
    def _():
        fire_pair(0, fk)

    @pl.when(jnp.logical_and(fired, par == 1))
    def _():
        fire_pair(1, fk)

    fk = jnp.where(fired, fk + 1, fk)
    par = jnp.where(fired, 1 - par, par)

    @pl.when(jnp.logical_and(fk > 0, par == 1))
    def _():
        retire(0)

    @pl.when(jnp.logical_and(fk > 0, par == 0))
    def _():
        retire(1)

    plsc.subcore_barrier()

    def scale_step(j, _):
        g_chunk = j * NS + s
        start_l = lax.min(g_chunk * CH, HALF - CH)
        gstart = base + start_l
        pltpu.sync_copy(acc_s.at[pl.ds(start_l, CH)], rows0.at[pl.ds(0, CH)])
        pltpu.sync_copy(s_h.at[pl.ds(gstart, CH)], rows0.at[pl.ds(32, CH)])
        pltpu.sync_copy(dinv_h.at[pl.ds(gstart, CH)], dinvbuf)

        def group(g, _):
            dvec = dinvbuf[pl.ds(g * LANES, LANES)]
            for r16 in range(LANES):
                d = _lane_bcast(dvec, r16)
                r = g * LANES + r16
                for k in range(EMB_DIM // LANES):
                    sl = pl.ds(k * LANES, LANES)
                    e = rows0[r, sl] * d
                    sv = rows0[32 + r, sl] + e
                    if last:
                        sv = sv * 0.25
                    rows0[64 + r, sl] = sv
                    if not last:
                        rows0[96 + r, sl] = e * d
            return 0

        lax.fori_loop(0, CH // LANES, group, 0)
        pltpu.sync_copy(rows0.at[pl.ds(64, CH)],
                        s_out_h.at[pl.ds(gstart, CH)])
        if not last:
            pltpu.sync_copy(rows0.at[pl.ds(96, CH)],
                            y_out_h.at[pl.ds(gstart, CH)])
        return 0

    lax.fori_loop(0, -(-HALF // (NS * CH)), scale_step, 0)


_deg_kernel = pl.kernel(
    _deg_kernel_body,
    out_type=[
        jax.ShapeDtypeStruct((N_NODES,), jnp.float32),
        jax.ShapeDtypeStruct((N_NODES, EMB_DIM), jnp.float32),
    ],
    mesh=_MESH,
    compiler_params=_PARAMS,
    scratch_types=[
        pltpu.VMEM((GI_D * DB,), jnp.int32),
        pltpu.VMEM((ACC_ROWS,), jnp.float32),
        pltpu.VMEM((SEG,), jnp.float32),
        pltpu.VMEM((SEG,), jnp.float32),
        pltpu.VMEM((CH,), jnp.float32),
        pltpu.VMEM((CH, EMB_DIM), jnp.float32),
        pltpu.VMEM((CH, EMB_DIM), jnp.float32),
        pltpu.VMEM_SHARED((NS, ACC_ROWS), jnp.float32),
    ],
)


def _layer_out_type(last):
    out = [jax.ShapeDtypeStruct((N_NODES, EMB_DIM), jnp.float32)]
    if not last:
        out.append(jax.ShapeDtypeStruct((N_NODES, EMB_DIM), jnp.float32))
    return out


def _layer_scratch():
    return [
        pltpu.VMEM((SCAN_B,), jnp.int32),
        pltpu.VMEM((SCAN_B,), jnp.int32),
        pltpu.VMEM((PEND,), jnp.int32),
        pltpu.VMEM((PEND,), jnp.int32),
        pltpu.VMEM((FB,), jnp.int32),
        pltpu.VMEM((FB,), jnp.int32),
        pltpu.VMEM((FB,), jnp.int32),
        pltpu.VMEM((FB,), jnp.int32),
        pltpu.VMEM((FB, EMB_DIM), jnp.float32),
        pltpu.VMEM((FB, EMB_DIM), jnp.float32),
        pltpu.VMEM((Z_CH, EMB_DIM), jnp.float32),
        pltpu.VMEM((CH,), jnp.float32),
        pltpu.VMEM_SHARED((ACC_ROWS, EMB_DIM), jnp.float32),
        pltpu.SemaphoreType.DMA,
        pltpu.SemaphoreType.DMA,
    ]


_mid_layer = pl.kernel(
    functools.partial(_layer_kernel_body, False),
    out_type=_layer_out_type(False),
    mesh=_MESH,
    compiler_params=_PARAMS,
    scratch_types=_layer_scratch(),
)

_last_layer = pl.kernel(
    functools.partial(_layer_kernel_body, True),
    out_type=_layer_out_type(True),
    mesh=_MESH,
    compiler_params=_PARAMS,
    scratch_types=_layer_scratch(),
)


def kernel(edge_index, edge_attrs, table):
    del edge_attrs
    pad = E_PAD - N_EDGES
    from_p = jnp.concatenate(
        [edge_index[0], jnp.zeros((pad,), jnp.int32)])
    to_p = jnp.concatenate(
        [edge_index[1], jnp.full((pad,), N_NODES, jnp.int32)])

    dinv, y = _deg_kernel(to_p, table)
    s = table
    for layer in range(N_LAYERS):
        if layer == N_LAYERS - 1:
            (s,) = _last_layer(from_p, to_p, y, s, dinv)
        else:
            s, y = _mid_layer(from_p, to_p, y, s, dinv)
    return (table, s)


# speedup vs baseline: 1.2575x; 1.2575x over previous
import functools

import jax
import jax.numpy as jnp
from jax import lax
from jax.experimental import pallas as pl
from jax.experimental.pallas import tpu as pltpu
from jax.experimental.pallas import tpu_sc as plsc

N_NODES = 50000
EMB_DIM = 64
N_EDGES = 800000
N_LAYERS = 3

NC = 2
NS = 16
LANES = 16

HALF = N_NODES // NC
ACC_ROWS = 25088
ROWS_PER_TILE = ACC_ROWS // NS
DUMMY = HALF

SCAN_B = 768
FB = 128
PEND = 160
EDGES_PER_TILE = 50688
SCAN_BLOCKS = EDGES_PER_TILE // SCAN_B
E_PAD = NS * EDGES_PER_TILE

DB = 128
DCHUNKS = E_PAD // (NS * DB)
GI_D = 4
DBLOCKS = DCHUNKS // GI_D

CH = 32
Z_CH = 32
SEG = ROWS_PER_TILE

_MESH = plsc.VectorSubcoreMesh(core_axis_name="c", subcore_axis_name="s")
_PARAMS = pltpu.CompilerParams(
    use_tc_tiling_on_sc=False, needs_layout_passes=False)


def _lane_bcast(vec, lane):
    idx = jnp.full((LANES,), lane, jnp.int32)
    return jnp.take_along_axis(vec, idx, axis=0)


def _rsqrt16(d):
    i = lax.bitcast_convert_type(d, jnp.int32)
    i = 0x5F3759DF - jnp.right_shift(i, 1)
    y = lax.bitcast_convert_type(i, jnp.float32)
    for _ in range(3):
        y = y * (1.5 - 0.5 * d * y * y)
    return jnp.where(d >= 0.5, y, 0.0)


def _remap16(t, base):
    local = t - base
    ok = jnp.logical_and(local >= 0, local < HALF)
    return jnp.where(ok, local, DUMMY)


def _deg_kernel_body(to_h, table_h, dinv_h, y0_h,
                     tob_big, hist_v, tmp_seg, seg_deg, dinvbuf, tbuf, ybuf,
                     stage_s):
    c = lax.axis_index("c")
    s = lax.axis_index("s")
    base = c * HALF
    zeros16 = jnp.zeros((LANES,), jnp.float32)
    ones16 = jnp.ones((LANES,), jnp.float32)

    def zfill(i, _):
        hist_v[pl.ds(i * LANES, LANES)] = zeros16
        return 0

    lax.fori_loop(0, ACC_ROWS // LANES, zfill, 0)

    def block_step(nb, _):
        ebase = (s * DCHUNKS + nb * GI_D) * DB
        pltpu.sync_copy(to_h.at[pl.ds(ebase, GI_D * DB)], tob_big)
        for q in range(GI_D * DB // LANES):
            t = tob_big[pl.ds(q * LANES, LANES)]
            plsc.addupdate_scatter(hist_v, [_remap16(t, base)], ones16)
        return 0

    lax.fori_loop(0, DBLOCKS, block_step, 0)

    pltpu.sync_copy(hist_v, stage_s.at[s])
    plsc.subcore_barrier()

    seg0 = s * SEG
    pltpu.sync_copy(stage_s.at[0, pl.ds(seg0, SEG)], seg_deg)
    for t in range(1, NS):
        pltpu.sync_copy(stage_s.at[t, pl.ds(seg0, SEG)], tmp_seg)

        def acc_step(i, _, t=t):
            sl = pl.ds(i * LANES, LANES)
            seg_deg[sl] = seg_deg[sl] + tmp_seg[sl]
            return 0

        lax.fori_loop(0, SEG // LANES, acc_step, 0)

    def scale_step(k, _):
        start_l = lax.min(seg0 + k * CH, HALF - CH)
        off = start_l - seg0
        gstart = base + start_l
        for g in range(CH // LANES):
            d = seg_deg[pl.ds(off + g * LANES, LANES)]
            dinvbuf[pl.ds(g * LANES, LANES)] = _rsqrt16(d)
        pltpu.sync_copy(dinvbuf, dinv_h.at[pl.ds(gstart, CH)])
        pltpu.sync_copy(table_h.at[pl.ds(gstart, CH)], tbuf)

        def group(g, _):
            dvec = dinvbuf[pl.ds(g * LANES, LANES)]
            for r16 in range(LANES):
                d = _lane_bcast(dvec, r16)
                r = g * LANES + r16
                for k2 in range(EMB_DIM // LANES):
                    sl = pl.ds(k2 * LANES, LANES)
                    ybuf[r, sl] = tbuf[r, sl] * d
            return 0

        lax.fori_loop(0, CH // LANES, group, 0)
        pltpu.sync_copy(ybuf, y0_h.at[pl.ds(gstart, CH)])
        return 0

    lax.fori_loop(0, SEG // CH, scale_step, 0)


def _layer_kernel_body(last, from_h, to_h, y_h, s_h, dinv_h, *refs):
    if last:
        (s_out_h, from_blk, to_blk, pf, pidx, gfid, sidx,
         rows0, zrow, dinvbuf, acc_s, gsem) = refs
        y_out_h = None
    else:
        (s_out_h, y_out_h, from_blk, to_blk, pf, pidx, gfid, sidx,
         rows0, zrow, dinvbuf, acc_s, gsem) = refs

    c = lax.axis_index("c")
    s = lax.axis_index("s")
    base = c * HALF
    zeros16 = jnp.zeros((LANES,), jnp.float32)

    def zfill(r, _):
        for k in range(EMB_DIM // LANES):
            zrow[r, pl.ds(k * LANES, LANES)] = zeros16
        return 0

    lax.fori_loop(0, Z_CH, zfill, 0)

    def zcopy(j, _):
        pltpu.sync_copy(zrow,
                        acc_s.at[pl.ds(s * ROWS_PER_TILE + j * Z_CH, Z_CH)])
        return 0

    lax.fori_loop(0, ROWS_PER_TILE // Z_CH, zcopy, 0)
    plsc.subcore_barrier()

    def fire():
        for k in range(FB // LANES):
            sl = pl.ds(k * LANES, LANES)
            gfid[sl] = pf[sl]
            sidx[sl] = pidx[sl]
        pltpu.async_copy(y_h.at[gfid], rows0, gsem).wait()
        pltpu.sync_copy(rows0, acc_s.at[sidx], add=True)

    def block_fn(nb, cnt):
        ebase = s * EDGES_PER_TILE + nb * SCAN_B
        pltpu.sync_copy(from_h.at[pl.ds(ebase, SCAN_B)], from_blk)
        pltpu.sync_copy(to_h.at[pl.ds(ebase, SCAN_B)], to_blk)
        for g in range(SCAN_B // LANES):
            sl = pl.ds(g * LANES, LANES)
            f16 = from_blk[sl]
            local = to_blk[sl] - base
            ok = jnp.logical_and(local >= 0, local < HALF)
            plsc.store_compressed(pf.at[pl.ds(cnt, LANES)], f16, mask=ok)
            plsc.store_compressed(pidx.at[pl.ds(cnt, LANES)], local, mask=ok)
            cnt = cnt + plsc.all_reduce_population_count(ok)[0]

            @pl.when(cnt >= FB)
            def _():
                fire()
                pf[pl.ds(0, LANES)] = pf[pl.ds(FB, LANES)]
                pidx[pl.ds(0, LANES)] = pidx[pl.ds(FB, LANES)]

            cnt = jnp.where(cnt >= FB, cnt - FB, cnt)
        return cnt

    cnt = lax.fori_loop(0, SCAN_BLOCKS, block_fn, jnp.int32(0))

    iota16 = lax.iota(jnp.int32, LANES)
    for k in range(FB // LANES):
        sl = pl.ds(k * LANES, LANES)
        m = (iota16 + k * LANES) < cnt
        pf[sl] = jnp.where(m, pf[sl], 0)
        pidx[sl] = jnp.where(m, pidx[sl], DUMMY)

    @pl.when(cnt > 0)
    def _():
        fire()

    plsc.subcore_barrier()

    def scale_step(j, _):
        g_chunk = j * NS + s
        start_l = lax.min(g_chunk * CH, HALF - CH)
        gstart = base + start_l
        pltpu.sync_copy(acc_s.at[pl.ds(start_l, CH)], rows0.at[pl.ds(0, CH)])
        pltpu.sync_copy(s_h.at[pl.ds(gstart, CH)], rows0.at[pl.ds(32, CH)])
        pltpu.sync_copy(dinv_h.at[pl.ds(gstart, CH)], dinvbuf)

        def group(g, _):
            dvec = dinvbuf[pl.ds(g * LANES, LANES)]
            for r16 in range(LANES):
                d = _lane_bcast(dvec, r16)
                r = g * LANES + r16
                for k in range(EMB_DIM // LANES):
                    sl = pl.ds(k * LANES, LANES)
                    e = rows0[r, sl] * d
                    sv = rows0[32 + r, sl] + e
                    if last:
                        sv = sv * 0.25
                    rows0[64 + r, sl] = sv
                    if not last:
                        rows0[96 + r, sl] = e * d
            return 0

        lax.fori_loop(0, CH // LANES, group, 0)
        pltpu.sync_copy(rows0.at[pl.ds(64, CH)],
                        s_out_h.at[pl.ds(gstart, CH)])
        if not last:
            pltpu.sync_copy(rows0.at[pl.ds(96, CH)],
                            y_out_h.at[pl.ds(gstart, CH)])
        return 0

    lax.fori_loop(0, -(-HALF // (NS * CH)), scale_step, 0)


_deg_kernel = pl.kernel(
    _deg_kernel_body,
    out_type=[
        jax.ShapeDtypeStruct((N_NODES,), jnp.float32),
        jax.ShapeDtypeStruct((N_NODES, EMB_DIM), jnp.float32),
    ],
    mesh=_MESH,
    compiler_params=_PARAMS,
    scratch_types=[
        pltpu.VMEM((GI_D * DB,), jnp.int32),
        pltpu.VMEM((ACC_ROWS,), jnp.float32),
        pltpu.VMEM((SEG,), jnp.float32),
        pltpu.VMEM((SEG,), jnp.float32),
        pltpu.VMEM((CH,), jnp.float32),
        pltpu.VMEM((CH, EMB_DIM), jnp.float32),
        pltpu.VMEM((CH, EMB_DIM), jnp.float32),
        pltpu.VMEM_SHARED((NS, ACC_ROWS), jnp.float32),
    ],
)


def _layer_out_type(last):
    out = [jax.ShapeDtypeStruct((N_NODES, EMB_DIM), jnp.float32)]
    if not last:
        out.append(jax.ShapeDtypeStruct((N_NODES, EMB_DIM), jnp.float32))
    return out


def _layer_scratch():
    return [
        pltpu.VMEM((SCAN_B,), jnp.int32),
        pltpu.VMEM((SCAN_B,), jnp.int32),
        pltpu.VMEM((PEND,), jnp.int32),
        pltpu.VMEM((PEND,), jnp.int32),
        pltpu.VMEM((FB,), jnp.int32),
        pltpu.VMEM((FB,), jnp.int32),
        pltpu.VMEM((FB, EMB_DIM), jnp.float32),
        pltpu.VMEM((Z_CH, EMB_DIM), jnp.float32),
        pltpu.VMEM((CH,), jnp.float32),
        pltpu.VMEM_SHARED((ACC_ROWS, EMB_DIM), jnp.float32),
        pltpu.SemaphoreType.DMA,
    ]


_mid_layer = pl.kernel(
    functools.partial(_layer_kernel_body, False),
    out_type=_layer_out_type(False),
    mesh=_MESH,
    compiler_params=_PARAMS,
    scratch_types=_layer_scratch(),
)

_last_layer = pl.kernel(
    functools.partial(_layer_kernel_body, True),
    out_type=_layer_out_type(True),
    mesh=_MESH,
    compiler_params=_PARAMS,
    scratch_types=_layer_scratch(),
)


def kernel(edge_index, edge_attrs, table):
    del edge_attrs
    pad = E_PAD - N_EDGES
    from_p = jnp.concatenate(
        [edge_index[0], jnp.zeros((pad,), jnp.int32)])
    to_p = jnp.concatenate(
        [edge_index[1], jnp.full((pad,), N_NODES, jnp.int32)])

    dinv, y = _deg_kernel(to_p, table)
    s = table
    for layer in range(N_LAYERS):
        if layer == N_LAYERS - 1:
            (s,) = _last_layer(from_p, to_p, y, s, dinv)
        else:
            s, y = _mid_layer(from_p, to_p, y, s, dinv)
    return (table, s)


# speedup vs baseline: 1.3014x; 1.0349x over previous
import functools

import jax
import jax.numpy as jnp
from jax import lax
from jax.experimental import pallas as pl
from jax.experimental.pallas import tpu as pltpu
from jax.experimental.pallas import tpu_sc as plsc

N_NODES = 50000
EMB_DIM = 64
N_EDGES = 800000
N_LAYERS = 3

NC = 2
NS = 16
LANES = 16

HALF = N_NODES // NC
ACC_ROWS = 25088
ROWS_PER_TILE = ACC_ROWS // NS
DUMMY = HALF

SCAN_B = 1536
FB = 128
PEND = 160
EDGES_PER_TILE = 50688
SCAN_BLOCKS = EDGES_PER_TILE // SCAN_B
E_PAD = NS * EDGES_PER_TILE

DB = 128
DCHUNKS = E_PAD // (NS * DB)
GI_D = 12
DBLOCKS = DCHUNKS // GI_D

CH = 64
DCH = 32
Z_CH = 32
SEG = ROWS_PER_TILE

_MESH = plsc.VectorSubcoreMesh(core_axis_name="c", subcore_axis_name="s")
_PARAMS = pltpu.CompilerParams(
    use_tc_tiling_on_sc=False, needs_layout_passes=False)


def _lane_bcast(vec, lane):
    idx = jnp.full((LANES,), lane, jnp.int32)
    return jnp.take_along_axis(vec, idx, axis=0)


def _rsqrt16(d):
    i = lax.bitcast_convert_type(d, jnp.int32)
    i = 0x5F3759DF - jnp.right_shift(i, 1)
    y = lax.bitcast_convert_type(i, jnp.float32)
    for _ in range(3):
        y = y * (1.5 - 0.5 * d * y * y)
    return jnp.where(d >= 0.5, y, 0.0)


def _remap16(t, base):
    local = t - base
    ok = jnp.logical_and(local >= 0, local < HALF)
    return jnp.where(ok, local, DUMMY)


def _deg_kernel_body(to_h, table_h, dinv_h, y0_h,
                     tob_big, hist_v, tmp_seg, seg_deg, dinvbuf, tbuf, ybuf,
                     stage_s):
    c = lax.axis_index("c")
    s = lax.axis_index("s")
    base = c * HALF
    zeros16 = jnp.zeros((LANES,), jnp.float32)
    ones16 = jnp.ones((LANES,), jnp.float32)

    def zfill(i, _):
        hist_v[pl.ds(i * LANES, LANES)] = zeros16
        return 0

    lax.fori_loop(0, ACC_ROWS // LANES, zfill, 0)

    def block_step(nb, _):
        ebase = (s * DCHUNKS + nb * GI_D) * DB
        pltpu.sync_copy(to_h.at[pl.ds(ebase, GI_D * DB)], tob_big)
        for q in range(GI_D * DB // LANES):
            t = tob_big[pl.ds(q * LANES, LANES)]
            plsc.addupdate_scatter(hist_v, [_remap16(t, base)], ones16)
        return 0

    lax.fori_loop(0, DBLOCKS, block_step, 0)

    pltpu.sync_copy(hist_v, stage_s.at[s])
    plsc.subcore_barrier()

    seg0 = s * SEG
    pltpu.sync_copy(stage_s.at[0, pl.ds(seg0, SEG)], seg_deg)
    for t in range(1, NS):
        pltpu.sync_copy(stage_s.at[t, pl.ds(seg0, SEG)], tmp_seg)

        def acc_step(i, _, t=t):
            sl = pl.ds(i * LANES, LANES)
            seg_deg[sl] = seg_deg[sl] + tmp_seg[sl]
            return 0

        lax.fori_loop(0, SEG // LANES, acc_step, 0)

    def scale_step(k, _):
        start_l = lax.min(seg0 + k * DCH, HALF - DCH)
        off = start_l - seg0
        gstart = base + start_l
        for g in range(DCH // LANES):
            d = seg_deg[pl.ds(off + g * LANES, LANES)]
            dinvbuf[pl.ds(g * LANES, LANES)] = _rsqrt16(d)
        pltpu.sync_copy(dinvbuf, dinv_h.at[pl.ds(gstart, DCH)])
        pltpu.sync_copy(table_h.at[pl.ds(gstart, DCH)], tbuf)

        def group(g, _):
            dvec = dinvbuf[pl.ds(g * LANES, LANES)]
            for r16 in range(LANES):
                d = _lane_bcast(dvec, r16)
                r = g * LANES + r16
                for k2 in range(EMB_DIM // LANES):
                    sl = pl.ds(k2 * LANES, LANES)
                    ybuf[r, sl] = tbuf[r, sl] * d
            return 0

        lax.fori_loop(0, DCH // LANES, group, 0)
        pltpu.sync_copy(ybuf, y0_h.at[pl.ds(gstart, DCH)])
        return 0

    lax.fori_loop(0, SEG // DCH, scale_step, 0)


def _layer_kernel_body(last, from_h, to_h, y_h, s_h, dinv_h, *refs):
    if last:
        (s_out_h, from_blk, to_blk, pf, pidx, gfid, sidx,
         rows0, zrow, obuf, ybuf, dinvbuf, acc_s, gsem) = refs
        y_out_h = None
    else:
        (s_out_h, y_out_h, from_blk, to_blk, pf, pidx, gfid, sidx,
         rows0, zrow, obuf, ybuf, dinvbuf, acc_s, gsem) = refs

    c = lax.axis_index("c")
    s = lax.axis_index("s")
    base = c * HALF
    zeros16 = jnp.zeros((LANES,), jnp.float32)

    def zfill(r, _):
        for k in range(EMB_DIM // LANES):
            zrow[r, pl.ds(k * LANES, LANES)] = zeros16
        return 0

    lax.fori_loop(0, Z_CH, zfill, 0)

    def zcopy(j, _):
        pltpu.sync_copy(zrow,
                        acc_s.at[pl.ds(s * ROWS_PER_TILE + j * Z_CH, Z_CH)])
        return 0

    lax.fori_loop(0, ROWS_PER_TILE // Z_CH, zcopy, 0)
    plsc.subcore_barrier()

    def fire():
        for k in range(FB // LANES):
            sl = pl.ds(k * LANES, LANES)
            gfid[sl] = pf[sl]
            sidx[sl] = pidx[sl]
        pltpu.async_copy(y_h.at[gfid], rows0, gsem).wait()
        pltpu.sync_copy(rows0, acc_s.at[sidx], add=True)

    def block_fn(nb, cnt):
        ebase = s * EDGES_PER_TILE + nb * SCAN_B
        pltpu.sync_copy(from_h.at[pl.ds(ebase, SCAN_B)], from_blk)
        pltpu.sync_copy(to_h.at[pl.ds(ebase, SCAN_B)], to_blk)
        for g in range(SCAN_B // LANES):
            sl = pl.ds(g * LANES, LANES)
            f16 = from_blk[sl]
            local = to_blk[sl] - base
            ok = jnp.logical_and(local >= 0, local < HALF)
            plsc.store_compressed(pf.at[pl.ds(cnt, LANES)], f16, mask=ok)
            plsc.store_compressed(pidx.at[pl.ds(cnt, LANES)], local, mask=ok)
            cnt = cnt + plsc.all_reduce_population_count(ok)[0]

            @pl.when(cnt >= FB)
            def _():
                fire()
                pf[pl.ds(0, LANES)] = pf[pl.ds(FB, LANES)]
                pidx[pl.ds(0, LANES)] = pidx[pl.ds(FB, LANES)]

            cnt = jnp.where(cnt >= FB, cnt - FB, cnt)
        return cnt

    cnt = lax.fori_loop(0, SCAN_BLOCKS, block_fn, jnp.int32(0))

    iota16 = lax.iota(jnp.int32, LANES)
    for k in range(FB // LANES):
        sl = pl.ds(k * LANES, LANES)
        m = (iota16 + k * LANES) < cnt
        pf[sl] = jnp.where(m, pf[sl], 0)
        pidx[sl] = jnp.where(m, pidx[sl], DUMMY)

    @pl.when(cnt > 0)
    def _():
        fire()

    plsc.subcore_barrier()

    def scale_step(j, _):
        g_chunk = j * NS + s
        start_l = lax.min(g_chunk * CH, HALF - CH)
        gstart = base + start_l
        pltpu.sync_copy(acc_s.at[pl.ds(start_l, CH)], rows0.at[pl.ds(0, CH)])
        pltpu.sync_copy(s_h.at[pl.ds(gstart, CH)], rows0.at[pl.ds(64, CH)])
        pltpu.sync_copy(dinv_h.at[pl.ds(gstart, CH)], dinvbuf)

        def group(g, _):
            dvec = dinvbuf[pl.ds(g * LANES, LANES)]
            for r16 in range(LANES):
                d = _lane_bcast(dvec, r16)
                r = g * LANES + r16
                for k in range(EMB_DIM // LANES):
                    sl = pl.ds(k * LANES, LANES)
                    e = rows0[r, sl] * d
                    sv = rows0[64 + r, sl] + e
                    if last:
                        sv = sv * 0.25
                    obuf[r, sl] = sv
                    if not last:
                        ybuf[r, sl] = e * d
            return 0

        lax.fori_loop(0, CH // LANES, group, 0)
        pltpu.sync_copy(obuf, s_out_h.at[pl.ds(gstart, CH)])
        if not last:
            pltpu.sync_copy(ybuf, y_out_h.at[pl.ds(gstart, CH)])
        return 0

    lax.fori_loop(0, -(-HALF // (NS * CH)), scale_step, 0)


_deg_kernel = pl.kernel(
    _deg_kernel_body,
    out_type=[
        jax.ShapeDtypeStruct((N_NODES,), jnp.float32),
        jax.ShapeDtypeStruct((N_NODES, EMB_DIM), jnp.float32),
    ],
    mesh=_MESH,
    compiler_params=_PARAMS,
    scratch_types=[
        pltpu.VMEM((GI_D * DB,), jnp.int32),
        pltpu.VMEM((ACC_ROWS,), jnp.float32),
        pltpu.VMEM((SEG,), jnp.float32),
        pltpu.VMEM((SEG,), jnp.float32),
        pltpu.VMEM((DCH,), jnp.float32),
        pltpu.VMEM((DCH, EMB_DIM), jnp.float32),
        pltpu.VMEM((DCH, EMB_DIM), jnp.float32),
        pltpu.VMEM_SHARED((NS, ACC_ROWS), jnp.float32),
    ],
)


def _layer_out_type(last):
    out = [jax.ShapeDtypeStruct((N_NODES, EMB_DIM), jnp.float32)]
    if not last:
        out.append(jax.ShapeDtypeStruct((N_NODES, EMB_DIM), jnp.float32))
    return out


def _layer_scratch():
    return [
        pltpu.VMEM((SCAN_B,), jnp.int32),
        pltpu.VMEM((SCAN_B,), jnp.int32),
        pltpu.VMEM((PEND,), jnp.int32),
        pltpu.VMEM((PEND,), jnp.int32),
        pltpu.VMEM((FB,), jnp.int32),
        pltpu.VMEM((FB,), jnp.int32),
        pltpu.VMEM((FB, EMB_DIM), jnp.float32),
        pltpu.VMEM((Z_CH, EMB_DIM), jnp.float32),
        pltpu.VMEM((CH, EMB_DIM), jnp.float32),
        pltpu.VMEM((CH, EMB_DIM), jnp.float32),
        pltpu.VMEM((CH,), jnp.float32),
        pltpu.VMEM_SHARED((ACC_ROWS, EMB_DIM), jnp.float32),
        pltpu.SemaphoreType.DMA,
    ]


_mid_layer = pl.kernel(
    functools.partial(_layer_kernel_body, False),
    out_type=_layer_out_type(False),
    mesh=_MESH,
    compiler_params=_PARAMS,
    scratch_types=_layer_scratch(),
)

_last_layer = pl.kernel(
    functools.partial(_layer_kernel_body, True),
    out_type=_layer_out_type(True),
    mesh=_MESH,
    compiler_params=_PARAMS,
    scratch_types=_layer_scratch(),
)


def kernel(edge_index, edge_attrs, table):
    del edge_attrs
    pad = E_PAD - N_EDGES
    from_p = jnp.concatenate(
        [edge_index[0], jnp.zeros((pad,), jnp.int32)])
    to_p = jnp.concatenate(
        [edge_index[1], jnp.full((pad,), N_NODES, jnp.int32)])

    dinv, y = _deg_kernel(to_p, table)
    s = table
    for layer in range(N_LAYERS):
        if layer == N_LAYERS - 1:
            (s,) = _last_layer(from_p, to_p, y, s, dinv)
        else:
            s, y = _mid_layer(from_p, to_p, y, s, dinv)
    return (table, s)
